# Initial kernel scaffold; baseline (speedup 1.0000x reference)
import jax
import jax.numpy as jnp
from jax.experimental import pallas as pl


def kernel(position_ids, emb_weight):
    raise NotImplementedError("write your pallas kernel here")


# speedup vs baseline: 2.1818x; 2.1818x over previous
import functools

import jax
import jax.numpy as jnp
from jax import lax
from jax.experimental import pallas as pl
from jax.experimental.pallas import tpu as pltpu
from jax.experimental.pallas import tpu_sc as plsc


def kernel(position_ids, emb_weight):
    B, S = position_ids.shape
    V, D = emb_weight.shape
    info = plsc.get_sparse_core_info()
    NC, NS = info.num_cores, info.num_subcores
    NW = NC * NS

    total = B * S
    per_w = total // NW
    C = 64
    n_chunks = per_w // C

    idx3 = position_ids.reshape(NW, n_chunks, C).astype(jnp.int32)

    mesh = plsc.VectorSubcoreMesh(core_axis_name="c", subcore_axis_name="s")

    @functools.partial(
        pl.kernel,
        mesh=mesh,
        out_type=jax.ShapeDtypeStruct((total, D), jnp.float32),
        scratch_types=[
            pltpu.VMEM((n_chunks, C), jnp.int32),
            pltpu.VMEM((C, D), jnp.float32),
            pltpu.SemaphoreType.DMA,
        ],
    )
    def emb_kernel(idx_hbm, table_hbm, out_hbm, idx_v, rows_v, gsem):
        wid = lax.axis_index("s") * NC + lax.axis_index("c")
        base = wid * per_w
        pltpu.sync_copy(idx_hbm.at[wid], idx_v)

        def body(j, carry):
            pltpu.async_copy(table_hbm.at[idx_v.at[j]], rows_v, gsem).wait()
            pltpu.sync_copy(rows_v, out_hbm.at[pl.ds(base + j * C, C)])
            return carry

        lax.fori_loop(0, n_chunks, body, 0)

    out = emb_kernel(idx3, emb_weight)
    return out.reshape(B, S, D)


# speedup vs baseline: 2.2819x; 1.0459x over previous
import functools

import jax
import jax.numpy as jnp
from jax import lax
from jax.experimental import pallas as pl
from jax.experimental.pallas import tpu as pltpu
from jax.experimental.pallas import tpu_sc as plsc


def kernel(position_ids, emb_weight):
    B, S = position_ids.shape
    V, D = emb_weight.shape
    info = plsc.get_sparse_core_info()
    NC, NS = info.num_cores, info.num_subcores
    NW = NC * NS

    total = B * S
    per_w = total // NW
    C = 32
    n_chunks = per_w // C
    n_pairs = n_chunks // 2

    idx3 = position_ids.reshape(NW, n_chunks, C).astype(jnp.int32)

    mesh = plsc.VectorSubcoreMesh(core_axis_name="c", subcore_axis_name="s")

    @functools.partial(
        pl.kernel,
        mesh=mesh,
        out_type=jax.ShapeDtypeStruct((total, D), jnp.float32),
        scratch_types=[
            pltpu.VMEM((n_chunks, C), jnp.int32),
            pltpu.VMEM((C, D), jnp.float32),
            pltpu.VMEM((C, D), jnp.float32),
            pltpu.SemaphoreType.DMA,
            pltpu.SemaphoreType.DMA,
        ],
    )
    def emb_kernel(idx_hbm, table_hbm, out_hbm, idx_v, rows0, rows1, g0, g1):
        wid = lax.axis_index("s") * NC + lax.axis_index("c")
        base = wid * per_w
        pltpu.sync_copy(idx_hbm.at[wid], idx_v)

        def start_gather(j, buf, sem):
            pltpu.async_copy(table_hbm.at[idx_v.at[j]], buf, sem)

        def wait_gather(j, buf, sem):
            pltpu.make_async_copy(table_hbm.at[idx_v.at[j]], buf, sem).wait()

        def write_out(j, buf):
            pltpu.sync_copy(buf, out_hbm.at[pl.ds(base + j * C, C)])

        start_gather(0, rows0, g0)

        def body(i, carry):
            j0 = 2 * i
            wait_gather(j0, rows0, g0)
            start_gather(j0 + 1, rows1, g1)
            write_out(j0, rows0)
            wait_gather(j0 + 1, rows1, g1)
            start_gather(j0 + 2, rows0, g0)
            write_out(j0 + 1, rows1)
            return carry

        lax.fori_loop(0, n_pairs - 1, body, 0)

        j0 = n_chunks - 2
        wait_gather(j0, rows0, g0)
        start_gather(j0 + 1, rows1, g1)
        write_out(j0, rows0)
        wait_gather(j0 + 1, rows1, g1)
        write_out(j0 + 1, rows1)

    out = emb_kernel(idx3, emb_weight)
    return out.reshape(B, S, D)


# speedup vs baseline: 2.2872x; 1.0023x over previous
import functools

import jax
import jax.numpy as jnp
from jax import lax
from jax.experimental import pallas as pl
from jax.experimental.pallas import tpu as pltpu
from jax.experimental.pallas import tpu_sc as plsc


def kernel(position_ids, emb_weight):
    B, S = position_ids.shape
    V, D = emb_weight.shape
    info = plsc.get_sparse_core_info()
    NC, NS = info.num_cores, info.num_subcores
    NW = NC * NS

    total = B * S
    per_w = total // NW
    C = 16
    NBUF = 4
    n_chunks = per_w // C
    rounds = n_chunks // NBUF

    idx3 = position_ids.reshape(NW, n_chunks, C).astype(jnp.int32)

    mesh = plsc.VectorSubcoreMesh(core_axis_name="c", subcore_axis_name="s")

    @functools.partial(
        pl.kernel,
        mesh=mesh,
        out_type=jax.ShapeDtypeStruct((total, D), jnp.float32),
        scratch_types=[
            pltpu.VMEM((n_chunks, C), jnp.int32),
        ]
        + [pltpu.VMEM((C, D), jnp.float32)] * NBUF
        + [pltpu.SemaphoreType.DMA] * (2 * NBUF),
    )
    def emb_kernel(idx_hbm, table_hbm, out_hbm, idx_v, *bufs_sems):
        bufs = bufs_sems[:NBUF]
        gsem = bufs_sems[NBUF:2 * NBUF]
        wsem = bufs_sems[2 * NBUF:]
        wid = lax.axis_index("s") * NC + lax.axis_index("c")
        base = wid * per_w
        pltpu.sync_copy(idx_hbm.at[wid], idx_v)

        def start_gather(j, b):
            pltpu.async_copy(table_hbm.at[idx_v.at[j]], bufs[b], gsem[b])

        def wait_gather(j, b):
            pltpu.make_async_copy(
                table_hbm.at[idx_v.at[j]], bufs[b], gsem[b]).wait()

        def start_write(j, b):
            pltpu.async_copy(
                bufs[b], out_hbm.at[pl.ds(base + j * C, C)], wsem[b])

        def wait_write(j, b):
            pltpu.make_async_copy(
                bufs[b], out_hbm.at[pl.ds(base + j * C, C)], wsem[b]).wait()

        for b in range(NBUF):
            start_gather(b, b)

        def body(i, carry):
            j0 = NBUF * i
            for b in range(NBUF):
                wait_gather(j0 + b, b)
                start_write(j0 + b, b)
            for b in range(NBUF):
                wait_write(j0 + b, b)
                start_gather(j0 + NBUF + b, b)
            return carry

        lax.fori_loop(0, rounds - 1, body, 0)

        j0 = NBUF * (rounds - 1)
        for b in range(NBUF):
            wait_gather(j0 + b, b)
            start_write(j0 + b, b)
        for b in range(NBUF):
            wait_write(j0 + b, b)

    out = emb_kernel(idx3, emb_weight)
    return out.reshape(B, S, D)


# speedup vs baseline: 2.3070x; 1.0086x over previous
import functools

import jax
import jax.numpy as jnp
from jax import lax
from jax.experimental import pallas as pl
from jax.experimental.pallas import tpu as pltpu
from jax.experimental.pallas import tpu_sc as plsc


def kernel(position_ids, emb_weight):
    B, S = position_ids.shape
    V, D = emb_weight.shape
    info = plsc.get_sparse_core_info()
    NC, NS = info.num_cores, info.num_subcores
    NW = NC * NS

    total = B * S
    per_w = total // NW
    C = 8
    NBUF = 8
    n_chunks = per_w // C
    rounds = n_chunks // NBUF

    idx3 = position_ids.reshape(NW, n_chunks, C).astype(jnp.int32)

    mesh = plsc.VectorSubcoreMesh(core_axis_name="c", subcore_axis_name="s")

    @functools.partial(
        pl.kernel,
        mesh=mesh,
        out_type=jax.ShapeDtypeStruct((total, D), jnp.float32),
        scratch_types=[
            pltpu.VMEM((n_chunks, C), jnp.int32),
        ]
        + [pltpu.VMEM((C, D), jnp.float32)] * NBUF
        + [pltpu.SemaphoreType.DMA] * (2 * NBUF),
    )
    def emb_kernel(idx_hbm, table_hbm, out_hbm, idx_v, *bufs_sems):
        bufs = bufs_sems[:NBUF]
        gsem = bufs_sems[NBUF:2 * NBUF]
        wsem = bufs_sems[2 * NBUF:]
        wid = lax.axis_index("s") * NC + lax.axis_index("c")
        base = wid * per_w
        pltpu.sync_copy(idx_hbm.at[wid], idx_v)

        def start_gather(j, b):
            pltpu.async_copy(table_hbm.at[idx_v.at[j]], bufs[b], gsem[b])

        def wait_gather(j, b):
            pltpu.make_async_copy(
                table_hbm.at[idx_v.at[j]], bufs[b], gsem[b]).wait()

        def start_write(j, b):
            pltpu.async_copy(
                bufs[b], out_hbm.at[pl.ds(base + j * C, C)], wsem[b])

        def wait_write(j, b):
            pltpu.make_async_copy(
                bufs[b], out_hbm.at[pl.ds(base + j * C, C)], wsem[b]).wait()

        for b in range(NBUF):
            start_gather(b, b)

        def body(i, carry):
            j0 = NBUF * i
            for b in range(NBUF):
                wait_gather(j0 + b, b)
                start_write(j0 + b, b)
            for b in range(NBUF):
                wait_write(j0 + b, b)
                start_gather(j0 + NBUF + b, b)
            return carry

        lax.fori_loop(0, rounds - 1, body, 0)

        j0 = NBUF * (rounds - 1)
        for b in range(NBUF):
            wait_gather(j0 + b, b)
            start_write(j0 + b, b)
        for b in range(NBUF):
            wait_write(j0 + b, b)

    out = emb_kernel(idx3, emb_weight)
    return out.reshape(B, S, D)
